# baseline (speedup 1.0000x reference)
import jax
import jax.numpy as jnp
from jax.experimental import pallas as pl
from jax.experimental.pallas import tpu as pltpu

_ROWS = 25088
_COLS = 1024
_NCHUNKS = 8
_CHUNK = _ROWS // _NCHUNKS


def _copy_dma(x_hbm, o_hbm, sems):
    copies = [
        pltpu.make_async_copy(
            x_hbm.at[pl.ds(i * _CHUNK, _CHUNK), :],
            o_hbm.at[pl.ds(i * _CHUNK, _CHUNK), :],
            sems.at[i],
        )
        for i in range(_NCHUNKS)
    ]
    for c in copies:
        c.start()
    for c in copies:
        c.wait()


def kernel(x):
    flat = x.reshape(_ROWS, _COLS)
    out = pl.pallas_call(
        _copy_dma,
        in_specs=[pl.BlockSpec(memory_space=pl.ANY)],
        out_specs=pl.BlockSpec(memory_space=pl.ANY),
        out_shape=jax.ShapeDtypeStruct((_ROWS, _COLS), x.dtype),
        scratch_shapes=[pltpu.SemaphoreType.DMA((_NCHUNKS,))],
    )(flat)
    return out.reshape(x.shape)


# speedup vs baseline: 8.2344x; 8.2344x over previous
import jax
import jax.numpy as jnp
from jax.experimental import pallas as pl

_B0 = 2


def _copy_block(x_ref, o_ref):
    o_ref[...] = x_ref[...]


def kernel(x):
    n0, n1, n2, n3 = x.shape
    return pl.pallas_call(
        _copy_block,
        grid=(n0 // _B0,),
        in_specs=[pl.BlockSpec((_B0, n1, n2, n3), lambda i: (i, 0, 0, 0))],
        out_specs=pl.BlockSpec((_B0, n1, n2, n3), lambda i: (i, 0, 0, 0)),
        out_shape=jax.ShapeDtypeStruct(x.shape, x.dtype),
    )(x)


# speedup vs baseline: 8.2379x; 1.0004x over previous
import jax
import jax.numpy as jnp
from jax.experimental import pallas as pl
from jax.experimental.pallas import tpu as pltpu

_ROWS = 458752
_COLS = 56
_CH = 8192
_NCHUNKS = _ROWS // _CH
_NBUF = 6


def _copy_body(x_hbm, o_hbm, buf, in_sems, out_sems):
    xf = x_hbm.reshape(_ROWS, _COLS)
    of = o_hbm.reshape(_ROWS, _COLS)

    def in_copy(i):
        return pltpu.make_async_copy(
            xf.at[pl.ds(i * _CH, _CH), :], buf.at[i % _NBUF], in_sems.at[i % _NBUF]
        )

    def out_copy(i):
        return pltpu.make_async_copy(
            buf.at[i % _NBUF], of.at[pl.ds(i * _CH, _CH), :], out_sems.at[i % _NBUF]
        )

    for i in range(min(_NBUF, _NCHUNKS)):
        in_copy(i).start()
    for i in range(_NCHUNKS):
        in_copy(i).wait()
        out_copy(i).start()
        nxt = i + _NBUF
        if nxt < _NCHUNKS:
            out_copy(i).wait()
            in_copy(nxt).start()
    for i in range(max(0, _NCHUNKS - _NBUF), _NCHUNKS):
        out_copy(i).wait()


def kernel(x):
    out = pl.pallas_call(
        _copy_body,
        in_specs=[pl.BlockSpec(memory_space=pl.ANY)],
        out_specs=pl.BlockSpec(memory_space=pl.ANY),
        out_shape=jax.ShapeDtypeStruct(x.shape, x.dtype),
        scratch_shapes=[
            pltpu.VMEM((_NBUF, _CH, _COLS), x.dtype),
            pltpu.SemaphoreType.DMA((_NBUF,)),
            pltpu.SemaphoreType.DMA((_NBUF,)),
        ],
    )(x)
    return out


# speedup vs baseline: 8.2709x; 1.0040x over previous
import jax
import jax.numpy as jnp
from jax.experimental import pallas as pl
from jax.experimental.pallas import tpu as pltpu

_ROWS = 458752
_COLS = 56
_CH = 8192
_NCHUNKS = _ROWS // _CH
_NBUF = 7


def _copy_body(x_hbm, o_hbm, *scratch):
    bufs = scratch[:_NBUF]
    in_sems = scratch[_NBUF:2 * _NBUF]
    out_sems = scratch[2 * _NBUF:]
    xf = x_hbm.reshape(_ROWS, _COLS)
    of = o_hbm.reshape(_ROWS, _COLS)

    def in_copy(i):
        s = i % _NBUF
        return pltpu.make_async_copy(
            xf.at[pl.ds(i * _CH, _CH), :], bufs[s], in_sems[s]
        )

    def out_copy(i):
        s = i % _NBUF
        return pltpu.make_async_copy(
            bufs[s], of.at[pl.ds(i * _CH, _CH), :], out_sems[s]
        )

    for i in range(min(_NBUF, _NCHUNKS)):
        in_copy(i).start()
    for i in range(_NCHUNKS):
        in_copy(i).wait()
        out_copy(i).start()
        nxt = i + _NBUF
        if nxt < _NCHUNKS:
            out_copy(i).wait()
            in_copy(nxt).start()
    for i in range(max(0, _NCHUNKS - _NBUF), _NCHUNKS):
        out_copy(i).wait()


def kernel(x):
    out = pl.pallas_call(
        _copy_body,
        in_specs=[pl.BlockSpec(memory_space=pl.ANY)],
        out_specs=pl.BlockSpec(memory_space=pl.ANY),
        out_shape=jax.ShapeDtypeStruct(x.shape, x.dtype),
        scratch_shapes=(
            [pltpu.VMEM((_CH, _COLS), jnp.float32) for _ in range(_NBUF)]
            + [pltpu.SemaphoreType.DMA(()) for _ in range(2 * _NBUF)]
        ),
    )(x)
    return out
